# Initial kernel scaffold; baseline (speedup 1.0000x reference)
import jax
import jax.numpy as jnp
from jax.experimental import pallas as pl


def kernel(node_feats, edge_index, edge_feats, Ws, bs, Wep, bep, Wg, bg, Wn1, bn1, Wn2, bn2, Wu1, bu1, Wu2, bu2, g_node, b_node, g_edge, b_edge):
    raise NotImplementedError("write your pallas kernel here")


# speedup vs baseline: 2.4668x; 2.4668x over previous
import functools

import jax
import jax.numpy as jnp
from jax import lax
from jax.experimental import pallas as pl
from jax.experimental.pallas import tpu as pltpu
from jax.experimental.pallas import tpu_sc as plsc


def _sc_gather(table, flat_idx):
    n_rows = flat_idx.shape[0]
    d = table.shape[1]
    info = plsc.get_sparse_core_info()
    nc, ns = info.num_cores, info.num_subcores
    nw = nc * ns
    per_w = n_rows // nw
    ch = 80
    n_ch = per_w // ch
    mesh = plsc.VectorSubcoreMesh(core_axis_name="c", subcore_axis_name="s")

    @functools.partial(
        pl.kernel,
        mesh=mesh,
        out_type=jax.ShapeDtypeStruct((n_rows, d), jnp.float32),
        scratch_types=[
            pltpu.VMEM((per_w,), jnp.int32),
            pltpu.VMEM((ch, d), jnp.float32),
            pltpu.SemaphoreType.DMA,
        ],
    )
    def k(table_hbm, idx_hbm, out_hbm, idx_v, rows_v, sem):
        wid = lax.axis_index("s") * nc + lax.axis_index("c")
        base = wid * per_w
        pltpu.sync_copy(idx_hbm.at[pl.ds(base, per_w)], idx_v)

        def body(j, carry):
            pltpu.async_copy(
                table_hbm.at[idx_v.at[pl.ds(j * ch, ch)]], rows_v, sem
            ).wait()
            pltpu.sync_copy(rows_v, out_hbm.at[pl.ds(base + j * ch, ch)])
            return carry

        lax.fori_loop(0, n_ch, body, 0)

    return k(table, flat_idx)


def _sc_scatter_add(msg, dst, zeros_nd):
    e, d = msg.shape
    n_pad = zeros_nd.shape[0]
    info = plsc.get_sparse_core_info()
    nc, ns = info.num_cores, info.num_subcores
    nw = nc * ns
    per_w = e // nw
    ch = 80
    n_ch = per_w // ch
    rows_per_tile = n_pad // ns
    mesh = plsc.VectorSubcoreMesh(core_axis_name="c", subcore_axis_name="s")

    @functools.partial(
        pl.kernel,
        mesh=mesh,
        out_type=jax.ShapeDtypeStruct((nc, n_pad, d), jnp.float32),
        scratch_types=[
            pltpu.VMEM((ch,), jnp.int32),
            pltpu.VMEM((ch, d), jnp.float32),
            pltpu.VMEM_SHARED((n_pad, d), jnp.float32),
            pltpu.SemaphoreType.DMA,
        ],
    )
    def k(msg_hbm, dst_hbm, zeros_hbm, out_hbm, idx_v, rows_v, acc_sh, sem):
        cid = lax.axis_index("c")
        sid = lax.axis_index("s")
        wid = sid * nc + cid
        pltpu.sync_copy(
            zeros_hbm.at[pl.ds(sid * rows_per_tile, rows_per_tile)],
            acc_sh.at[pl.ds(sid * rows_per_tile, rows_per_tile)],
        )
        plsc.subcore_barrier()

        base = wid * per_w

        def body(j, carry):
            pltpu.sync_copy(dst_hbm.at[pl.ds(base + j * ch, ch)], idx_v)
            pltpu.sync_copy(msg_hbm.at[pl.ds(base + j * ch, ch)], rows_v)
            pltpu.sync_copy(rows_v, acc_sh.at[idx_v], add=True)
            return carry

        lax.fori_loop(0, n_ch, body, 0)
        plsc.subcore_barrier()
        pltpu.sync_copy(
            acc_sh.at[pl.ds(sid * rows_per_tile, rows_per_tile)],
            out_hbm.at[cid].at[pl.ds(sid * rows_per_tile, rows_per_tile)],
        )

    return k(msg, dst, zeros_nd)


def _ln(x, g, b):
    m = jnp.mean(x, axis=-1, keepdims=True)
    v = jnp.mean((x - m) ** 2, axis=-1, keepdims=True)
    return (x - m) * lax.rsqrt(v + 1e-5) * g + b


def _edge_body(hs_ref, hd_ref, ef_ref, Wg_ref, bg_ref, Ws_ref, Wep_ref,
               bm_ref, Wu1_ref, bu1_ref, Wu2_ref, bu2_ref, ge_ref, be_ref,
               msg_ref, eo_ref):
    hs = hs_ref[...]
    hd = hd_ref[...]
    ef = ef_ref[...]
    x3 = jnp.concatenate([hs, hd, ef], axis=1)
    gate = jax.nn.sigmoid(
        jnp.dot(x3, Wg_ref[...], preferred_element_type=jnp.float32)
        + bg_ref[...]
    )
    lin = (
        jnp.dot(hs, Ws_ref[...], preferred_element_type=jnp.float32)
        + jnp.dot(ef, Wep_ref[...], preferred_element_type=jnp.float32)
        + bm_ref[...]
    )
    msg_ref[...] = gate * lin
    u1 = jax.nn.silu(
        jnp.dot(x3, Wu1_ref[...], preferred_element_type=jnp.float32)
        + bu1_ref[...]
    )
    eu = jnp.dot(u1, Wu2_ref[...], preferred_element_type=jnp.float32) + bu2_ref[...]
    eo_ref[...] = _ln(ef + eu, ge_ref[...], be_ref[...])


def _tc_edge(hs, hd, ef, Wg, bg, Ws, Wep, bm, Wu1, bu1, Wu2, bu2, ge, be):
    e, d = ef.shape
    be_blk = 2000
    grid = (e // be_blk,)
    row_spec = pl.BlockSpec((be_blk, d), lambda i: (i, 0))
    full = lambda a: pl.BlockSpec(a.shape, lambda i: (0,) * a.ndim)
    return pl.pallas_call(
        _edge_body,
        grid=grid,
        in_specs=[row_spec, row_spec, row_spec,
                  full(Wg), full(bg), full(Ws), full(Wep), full(bm),
                  full(Wu1), full(bu1), full(Wu2), full(bu2),
                  full(ge), full(be)],
        out_specs=[row_spec, row_spec],
        out_shape=[jax.ShapeDtypeStruct((e, d), jnp.float32),
                   jax.ShapeDtypeStruct((e, d), jnp.float32)],
    )(hs, hd, ef, Wg, bg, Ws, Wep, bm, Wu1, bu1, Wu2, bu2, ge, be)


def _node_body(nf_ref, p0_ref, p1_ref, W1a_ref, W1b_ref, b1_ref, W2_ref,
               b2_ref, gn_ref, bn_ref, out_ref):
    nf = nf_ref[...]
    agg = p0_ref[...] + p1_ref[...]
    h1 = jax.nn.silu(
        jnp.dot(nf, W1a_ref[...], preferred_element_type=jnp.float32)
        + jnp.dot(agg, W1b_ref[...], preferred_element_type=jnp.float32)
        + b1_ref[...]
    )
    h = jnp.dot(h1, W2_ref[...], preferred_element_type=jnp.float32) + b2_ref[...]
    out_ref[...] = _ln(nf + h, gn_ref[...], bn_ref[...])


def _tc_node(nf, p0, p1, W1a, W1b, b1, W2, b2, gn, bn):
    n, d = nf.shape
    bn_blk = 1000
    grid = (n // bn_blk,)
    row_spec = pl.BlockSpec((bn_blk, d), lambda i: (i, 0))
    full = lambda a: pl.BlockSpec(a.shape, lambda i: (0,) * a.ndim)
    return pl.pallas_call(
        _node_body,
        grid=grid,
        in_specs=[row_spec, row_spec, row_spec,
                  full(W1a), full(W1b), full(b1), full(W2), full(b2),
                  full(gn), full(bn)],
        out_specs=row_spec,
        out_shape=jax.ShapeDtypeStruct((n, d), jnp.float32),
    )(nf, p0, p1, W1a, W1b, b1, W2, b2, gn, bn)


def kernel(node_feats, edge_index, edge_feats, Ws, bs, Wep, bep, Wg, bg,
           Wn1, bn1, Wn2, bn2, Wu1, bu1, Wu2, bu2, g_node, b_node,
           g_edge, b_edge):
    n, d = node_feats.shape
    e = edge_feats.shape[0]

    flat_idx = edge_index.reshape(2 * e).astype(jnp.int32)
    gathered = _sc_gather(node_feats, flat_idx)
    hs = gathered[:e]
    hd = gathered[e:]

    row = lambda v: v.reshape(1, d)
    msg, edge_out = _tc_edge(
        hs, hd, edge_feats,
        Wg, row(bg), Ws, Wep, row(bs + bep),
        Wu1, row(bu1), Wu2, row(bu2), row(g_edge), row(b_edge),
    )

    dst = edge_index[1].astype(jnp.int32)
    n_pad = ((n + 127) // 128) * 128
    partials = _sc_scatter_add(msg, dst, jnp.zeros((n_pad, d), jnp.float32))

    node_out = _tc_node(
        node_feats, partials[0, :n], partials[1, :n],
        Wn1[:d], Wn1[d:], row(bn1), Wn2, row(bn2),
        row(g_node), row(b_node),
    )
    return (node_out, edge_out)


# speedup vs baseline: 2.8821x; 1.1684x over previous
import functools

import jax
import jax.numpy as jnp
from jax import lax
from jax.experimental import pallas as pl
from jax.experimental.pallas import tpu as pltpu
from jax.experimental.pallas import tpu_sc as plsc


def _sc_gather(table, flat_idx):
    n_rows = flat_idx.shape[0]
    d = table.shape[1]
    info = plsc.get_sparse_core_info()
    nc, ns = info.num_cores, info.num_subcores
    nw = nc * ns
    per_w = n_rows // nw
    ch = 80
    n_ch = per_w // ch
    nbuf = 5
    n_grp = n_ch // nbuf
    mesh = plsc.VectorSubcoreMesh(core_axis_name="c", subcore_axis_name="s")

    @functools.partial(
        pl.kernel,
        mesh=mesh,
        out_type=jax.ShapeDtypeStruct((n_rows, d), jnp.float32),
        scratch_types=[
            pltpu.VMEM((per_w,), jnp.int32),
            pltpu.VMEM((nbuf, ch, d), jnp.float32),
        ] + [pltpu.SemaphoreType.DMA] * (2 * nbuf),
    )
    def k(table_hbm, idx_hbm, out_hbm, idx_v, rows_v, *sems):
        sg, sp = sems[:nbuf], sems[nbuf:]
        wid = lax.axis_index("s") * nc + lax.axis_index("c")
        base = wid * per_w
        pltpu.sync_copy(idx_hbm.at[pl.ds(base, per_w)], idx_v)

        def gather(j, b):
            return pltpu.make_async_copy(
                table_hbm.at[idx_v.at[pl.ds(j * ch, ch)]], rows_v.at[b], sg[b]
            )

        def put(j, b):
            return pltpu.make_async_copy(
                rows_v.at[b], out_hbm.at[pl.ds(base + j * ch, ch)], sp[b]
            )

        for b in range(nbuf):
            gather(b, b).start()

        def body(g, carry):
            j0 = g * nbuf
            for b in range(nbuf):
                gather(j0 + b, b).wait()
                put(j0 + b, b).start()
            for b in range(nbuf):
                put(j0 + b, b).wait()

                @pl.when(g + 1 < n_grp)
                def _():
                    gather(j0 + nbuf + b, b).start()

            return carry

        lax.fori_loop(0, n_grp, body, 0)

    return k(table, flat_idx)


def _sc_scatter_add(msg, dst, zeros_nd):
    e, d = msg.shape
    n_pad = zeros_nd.shape[0]
    info = plsc.get_sparse_core_info()
    nc, ns = info.num_cores, info.num_subcores
    nw = nc * ns
    per_w = e // nw
    ch = 80
    n_ch = per_w // ch
    rows_per_tile = n_pad // ns
    mesh = plsc.VectorSubcoreMesh(core_axis_name="c", subcore_axis_name="s")

    @functools.partial(
        pl.kernel,
        mesh=mesh,
        out_type=jax.ShapeDtypeStruct((nc, n_pad, d), jnp.float32),
        scratch_types=[
            pltpu.VMEM((2, ch), jnp.int32),
            pltpu.VMEM((2, ch, d), jnp.float32),
            pltpu.VMEM_SHARED((n_pad, d), jnp.float32),
            pltpu.SemaphoreType.DMA,
            pltpu.SemaphoreType.DMA,
        ],
    )
    def k(msg_hbm, dst_hbm, zeros_hbm, out_hbm, idx_v, rows_v, acc_sh,
          sl0, sl1):
        sl = (sl0, sl1)
        cid = lax.axis_index("c")
        sid = lax.axis_index("s")
        wid = sid * nc + cid
        pltpu.sync_copy(
            zeros_hbm.at[pl.ds(sid * rows_per_tile, rows_per_tile)],
            acc_sh.at[pl.ds(sid * rows_per_tile, rows_per_tile)],
        )
        plsc.subcore_barrier()

        base = wid * per_w

        def body(j, carry):
            pltpu.sync_copy(dst_hbm.at[pl.ds(base + j * ch, ch)], idx_v.at[0])
            pltpu.sync_copy(msg_hbm.at[pl.ds(base + j * ch, ch)], rows_v.at[0])
            pltpu.sync_copy(rows_v.at[0], acc_sh.at[idx_v.at[0]], add=True)
            return carry

        lax.fori_loop(0, n_ch, body, 0)
        plsc.subcore_barrier()
        pltpu.sync_copy(
            acc_sh.at[pl.ds(sid * rows_per_tile, rows_per_tile)],
            out_hbm.at[cid].at[pl.ds(sid * rows_per_tile, rows_per_tile)],
        )

    return k(msg, dst, zeros_nd)


def _ln(x, g, b):
    m = jnp.mean(x, axis=-1, keepdims=True)
    v = jnp.mean((x - m) ** 2, axis=-1, keepdims=True)
    return (x - m) * lax.rsqrt(v + 1e-5) * g + b


def _edge_body(hs_ref, hd_ref, ef_ref, Wg_ref, bg_ref, Ws_ref, Wep_ref,
               bm_ref, Wu1_ref, bu1_ref, Wu2_ref, bu2_ref, ge_ref, be_ref,
               msg_ref, eo_ref):
    hs = hs_ref[...]
    hd = hd_ref[...]
    ef = ef_ref[...]
    x3 = jnp.concatenate([hs, hd, ef], axis=1)
    gate = jax.nn.sigmoid(
        jnp.dot(x3, Wg_ref[...], preferred_element_type=jnp.float32)
        + bg_ref[...]
    )
    lin = (
        jnp.dot(hs, Ws_ref[...], preferred_element_type=jnp.float32)
        + jnp.dot(ef, Wep_ref[...], preferred_element_type=jnp.float32)
        + bm_ref[...]
    )
    msg_ref[...] = gate * lin
    u1 = jax.nn.silu(
        jnp.dot(x3, Wu1_ref[...], preferred_element_type=jnp.float32)
        + bu1_ref[...]
    )
    eu = jnp.dot(u1, Wu2_ref[...], preferred_element_type=jnp.float32) + bu2_ref[...]
    eo_ref[...] = _ln(ef + eu, ge_ref[...], be_ref[...])


def _tc_edge(hs, hd, ef, Wg, bg, Ws, Wep, bm, Wu1, bu1, Wu2, bu2, ge, be):
    e, d = ef.shape
    be_blk = 2000
    grid = (e // be_blk,)
    row_spec = pl.BlockSpec((be_blk, d), lambda i: (i, 0))
    full = lambda a: pl.BlockSpec(a.shape, lambda i: (0,) * a.ndim)
    return pl.pallas_call(
        _edge_body,
        grid=grid,
        in_specs=[row_spec, row_spec, row_spec,
                  full(Wg), full(bg), full(Ws), full(Wep), full(bm),
                  full(Wu1), full(bu1), full(Wu2), full(bu2),
                  full(ge), full(be)],
        out_specs=[row_spec, row_spec],
        out_shape=[jax.ShapeDtypeStruct((e, d), jnp.float32),
                   jax.ShapeDtypeStruct((e, d), jnp.float32)],
    )(hs, hd, ef, Wg, bg, Ws, Wep, bm, Wu1, bu1, Wu2, bu2, ge, be)


def _node_body(nf_ref, p0_ref, p1_ref, W1a_ref, W1b_ref, b1_ref, W2_ref,
               b2_ref, gn_ref, bn_ref, out_ref):
    nf = nf_ref[...]
    agg = p0_ref[...] + p1_ref[...]
    h1 = jax.nn.silu(
        jnp.dot(nf, W1a_ref[...], preferred_element_type=jnp.float32)
        + jnp.dot(agg, W1b_ref[...], preferred_element_type=jnp.float32)
        + b1_ref[...]
    )
    h = jnp.dot(h1, W2_ref[...], preferred_element_type=jnp.float32) + b2_ref[...]
    out_ref[...] = _ln(nf + h, gn_ref[...], bn_ref[...])


def _tc_node(nf, p0, p1, W1a, W1b, b1, W2, b2, gn, bn):
    n, d = nf.shape
    bn_blk = 1000
    grid = (n // bn_blk,)
    row_spec = pl.BlockSpec((bn_blk, d), lambda i: (i, 0))
    full = lambda a: pl.BlockSpec(a.shape, lambda i: (0,) * a.ndim)
    return pl.pallas_call(
        _node_body,
        grid=grid,
        in_specs=[row_spec, row_spec, row_spec,
                  full(W1a), full(W1b), full(b1), full(W2), full(b2),
                  full(gn), full(bn)],
        out_specs=row_spec,
        out_shape=jax.ShapeDtypeStruct((n, d), jnp.float32),
    )(nf, p0, p1, W1a, W1b, b1, W2, b2, gn, bn)


def kernel(node_feats, edge_index, edge_feats, Ws, bs, Wep, bep, Wg, bg,
           Wn1, bn1, Wn2, bn2, Wu1, bu1, Wu2, bu2, g_node, b_node,
           g_edge, b_edge):
    n, d = node_feats.shape
    e = edge_feats.shape[0]

    flat_idx = edge_index.reshape(2 * e).astype(jnp.int32)
    gathered = _sc_gather(node_feats, flat_idx)
    hs = gathered[:e]
    hd = gathered[e:]

    row = lambda v: v.reshape(1, d)
    msg, edge_out = _tc_edge(
        hs, hd, edge_feats,
        Wg, row(bg), Ws, Wep, row(bs + bep),
        Wu1, row(bu1), Wu2, row(bu2), row(g_edge), row(b_edge),
    )

    dst = edge_index[1].astype(jnp.int32)
    n_pad = ((n + 127) // 128) * 128
    partials = _sc_scatter_add(msg, dst, jnp.zeros((n_pad, d), jnp.float32))

    node_out = _tc_node(
        node_feats, partials[0, :n], partials[1, :n],
        Wn1[:d], Wn1[d:], row(bn1), Wn2, row(bn2),
        row(g_node), row(b_node),
    )
    return (node_out, edge_out)


# speedup vs baseline: 3.5302x; 1.2248x over previous
import functools

import jax
import jax.numpy as jnp
from jax import lax
from jax.experimental import pallas as pl
from jax.experimental.pallas import tpu as pltpu
from jax.experimental.pallas import tpu_sc as plsc


def _sc_gather(table, flat_idx):
    n_rows = flat_idx.shape[0]
    d = table.shape[1]
    info = plsc.get_sparse_core_info()
    nc, ns = info.num_cores, info.num_subcores
    nw = nc * ns
    per_w = n_rows // nw
    ch = 80
    n_ch = per_w // ch
    nbuf = 5
    n_grp = n_ch // nbuf
    mesh = plsc.VectorSubcoreMesh(core_axis_name="c", subcore_axis_name="s")

    @functools.partial(
        pl.kernel,
        mesh=mesh,
        out_type=jax.ShapeDtypeStruct((n_rows, d), jnp.float32),
        scratch_types=[
            pltpu.VMEM((per_w,), jnp.int32),
            pltpu.VMEM((nbuf, ch, d), jnp.float32),
        ] + [pltpu.SemaphoreType.DMA] * (2 * nbuf),
    )
    def k(table_hbm, idx_hbm, out_hbm, idx_v, rows_v, *sems):
        sg, sp = sems[:nbuf], sems[nbuf:]
        wid = lax.axis_index("s") * nc + lax.axis_index("c")
        base = wid * per_w
        pltpu.sync_copy(idx_hbm.at[pl.ds(base, per_w)], idx_v)

        def gather(j, b):
            return pltpu.make_async_copy(
                table_hbm.at[idx_v.at[pl.ds(j * ch, ch)]], rows_v.at[b], sg[b]
            )

        def put(j, b):
            return pltpu.make_async_copy(
                rows_v.at[b], out_hbm.at[pl.ds(base + j * ch, ch)], sp[b]
            )

        for b in range(nbuf):
            gather(b, b).start()

        def body(g, carry):
            j0 = g * nbuf
            for b in range(nbuf):
                gather(j0 + b, b).wait()
                put(j0 + b, b).start()
            for b in range(nbuf):
                put(j0 + b, b).wait()

                @pl.when(g + 1 < n_grp)
                def _():
                    gather(j0 + nbuf + b, b).start()

            return carry

        lax.fori_loop(0, n_grp, body, 0)

    return k(table, flat_idx)


def _sc_scatter_add(msg, dst, zeros_nd):
    e, d = msg.shape
    n_pad = zeros_nd.shape[0]
    info = plsc.get_sparse_core_info()
    nc, ns = info.num_cores, info.num_subcores
    nw = nc * ns
    per_w = e // nw
    ch = 80
    n_ch = per_w // ch
    rows_per_tile = n_pad // ns
    mesh = plsc.VectorSubcoreMesh(core_axis_name="c", subcore_axis_name="s")

    @functools.partial(
        pl.kernel,
        mesh=mesh,
        out_type=jax.ShapeDtypeStruct((nc, n_pad, d), jnp.float32),
        scratch_types=[
            pltpu.VMEM((ch,), jnp.int32),
            pltpu.VMEM((ch, d), jnp.float32),
            pltpu.VMEM_SHARED((n_pad, d), jnp.float32),
        ],
    )
    def k(msg_hbm, dst_hbm, zeros_hbm, out_hbm, idx_v, rows_v, acc_sh):
        cid = lax.axis_index("c")
        sid = lax.axis_index("s")
        wid = sid * nc + cid
        pltpu.sync_copy(
            zeros_hbm.at[pl.ds(sid * rows_per_tile, rows_per_tile)],
            acc_sh.at[pl.ds(sid * rows_per_tile, rows_per_tile)],
        )
        plsc.subcore_barrier()

        base = wid * per_w

        def body(j, carry):
            pltpu.sync_copy(dst_hbm.at[pl.ds(base + j * ch, ch)], idx_v)
            pltpu.sync_copy(msg_hbm.at[pl.ds(base + j * ch, ch)], rows_v)
            pltpu.sync_copy(rows_v, acc_sh.at[idx_v], add=True)
            return carry

        lax.fori_loop(0, n_ch, body, 0)
        plsc.subcore_barrier()
        pltpu.sync_copy(
            acc_sh.at[pl.ds(sid * rows_per_tile, rows_per_tile)],
            out_hbm.at[cid].at[pl.ds(sid * rows_per_tile, rows_per_tile)],
        )

    return k(msg, dst, zeros_nd)


def _ln(x, g, b):
    m = jnp.mean(x, axis=-1, keepdims=True)
    v = jnp.mean((x - m) ** 2, axis=-1, keepdims=True)
    return (x - m) * lax.rsqrt(v + 1e-5) * g + b


def _dot(a, b):
    return jnp.dot(a, b, preferred_element_type=jnp.float32)


def _edge_body(hs_ref, hd_ref, ef_ref, Wgs_ref, Wgd_ref, Wge_ref, bg_ref,
               Ws_ref, Wep_ref, bm_ref, Wus_ref, Wud_ref, Wue_ref, bu1_ref,
               Wu2_ref, bu2_ref, ge_ref, be_ref, msg_ref, eo_ref):
    hs = hs_ref[...]
    hd = hd_ref[...]
    ef = ef_ref[...]
    gate = jax.nn.sigmoid(
        _dot(hs, Wgs_ref[...]) + _dot(hd, Wgd_ref[...])
        + _dot(ef, Wge_ref[...]) + bg_ref[...]
    )
    lin = _dot(hs, Ws_ref[...]) + _dot(ef, Wep_ref[...]) + bm_ref[...]
    msg_ref[...] = gate * lin
    u1 = jax.nn.silu(
        _dot(hs, Wus_ref[...]) + _dot(hd, Wud_ref[...])
        + _dot(ef, Wue_ref[...]) + bu1_ref[...]
    )
    eu = _dot(u1, Wu2_ref[...]) + bu2_ref[...]
    eo_ref[...] = _ln(ef + eu, ge_ref[...], be_ref[...])


def _tc_edge(gathered, ef, Wg, bg, Ws, Wep, bm, Wu1, bu1, Wu2, bu2, ge, be):
    e, d = ef.shape
    be_blk = 2000
    grid = (e // be_blk,)
    nblk = e // be_blk
    row_spec = pl.BlockSpec((be_blk, d), lambda i: (i, 0))
    hd_spec = pl.BlockSpec((be_blk, d), lambda i: (nblk + i, 0))
    full = lambda a: pl.BlockSpec(a.shape, lambda i: (0,) * a.ndim)
    ws = [Wg[:d], Wg[d:2 * d], Wg[2 * d:], bg, Ws, Wep, bm,
          Wu1[:d], Wu1[d:2 * d], Wu1[2 * d:], bu1, Wu2, bu2, ge, be]
    return pl.pallas_call(
        _edge_body,
        grid=grid,
        in_specs=[row_spec, hd_spec, row_spec] + [full(w) for w in ws],
        out_specs=[row_spec, row_spec],
        out_shape=[jax.ShapeDtypeStruct((e, d), jnp.float32),
                   jax.ShapeDtypeStruct((e, d), jnp.float32)],
    )(gathered, gathered, ef, *ws)


def _node_body(nf_ref, p0_ref, p1_ref, W1a_ref, W1b_ref, b1_ref, W2_ref,
               b2_ref, gn_ref, bn_ref, out_ref):
    nf = nf_ref[...]
    agg = p0_ref[...] + p1_ref[...]
    h1 = jax.nn.silu(
        jnp.dot(nf, W1a_ref[...], preferred_element_type=jnp.float32)
        + jnp.dot(agg, W1b_ref[...], preferred_element_type=jnp.float32)
        + b1_ref[...]
    )
    h = jnp.dot(h1, W2_ref[...], preferred_element_type=jnp.float32) + b2_ref[...]
    out_ref[...] = _ln(nf + h, gn_ref[...], bn_ref[...])


def _tc_node(nf, p0, p1, W1a, W1b, b1, W2, b2, gn, bn):
    n, d = nf.shape
    bn_blk = 1000
    grid = (n // bn_blk,)
    row_spec = pl.BlockSpec((bn_blk, d), lambda i: (i, 0))
    full = lambda a: pl.BlockSpec(a.shape, lambda i: (0,) * a.ndim)
    return pl.pallas_call(
        _node_body,
        grid=grid,
        in_specs=[row_spec, row_spec, row_spec,
                  full(W1a), full(W1b), full(b1), full(W2), full(b2),
                  full(gn), full(bn)],
        out_specs=row_spec,
        out_shape=jax.ShapeDtypeStruct((n, d), jnp.float32),
    )(nf, p0, p1, W1a, W1b, b1, W2, b2, gn, bn)


def kernel(node_feats, edge_index, edge_feats, Ws, bs, Wep, bep, Wg, bg,
           Wn1, bn1, Wn2, bn2, Wu1, bu1, Wu2, bu2, g_node, b_node,
           g_edge, b_edge):
    n, d = node_feats.shape
    e = edge_feats.shape[0]

    flat_idx = edge_index.reshape(2 * e).astype(jnp.int32)
    gathered = _sc_gather(node_feats, flat_idx)

    row = lambda v: v.reshape(1, d)
    msg, edge_out = _tc_edge(
        gathered, edge_feats,
        Wg, row(bg), Ws, Wep, row(bs + bep),
        Wu1, row(bu1), Wu2, row(bu2), row(g_edge), row(b_edge),
    )

    dst = edge_index[1].astype(jnp.int32)
    n_pad = ((n + 127) // 128) * 128
    partials = _sc_scatter_add(msg, dst, jnp.zeros((n_pad, d), jnp.float32))

    node_out = _tc_node(
        node_feats, partials[0, :n], partials[1, :n],
        Wn1[:d], Wn1[d:], row(bn1), Wn2, row(bn2),
        row(g_node), row(b_node),
    )
    return (node_out, edge_out)


# speedup vs baseline: 4.3796x; 1.2406x over previous
import functools

import jax
import jax.numpy as jnp
from jax import lax
from jax.experimental import pallas as pl
from jax.experimental.pallas import tpu as pltpu
from jax.experimental.pallas import tpu_sc as plsc


def _sc_gather(table, flat_idx):
    n_rows = flat_idx.shape[0]
    d = table.shape[1]
    dt = table.dtype
    info = plsc.get_sparse_core_info()
    nc, ns = info.num_cores, info.num_subcores
    nw = nc * ns
    per_w = n_rows // nw
    ch = 80
    n_ch = per_w // ch
    nbuf = 5
    n_grp = n_ch // nbuf
    mesh = plsc.VectorSubcoreMesh(core_axis_name="c", subcore_axis_name="s")

    @functools.partial(
        pl.kernel,
        mesh=mesh,
        out_type=jax.ShapeDtypeStruct((n_rows, d), dt),
        scratch_types=[
            pltpu.VMEM((per_w,), jnp.int32),
            pltpu.VMEM((nbuf, ch, d), dt),
        ] + [pltpu.SemaphoreType.DMA] * (2 * nbuf),
    )
    def k(table_hbm, idx_hbm, out_hbm, idx_v, rows_v, *sems):
        sg, sp = sems[:nbuf], sems[nbuf:]
        wid = lax.axis_index("s") * nc + lax.axis_index("c")
        base = wid * per_w
        pltpu.sync_copy(idx_hbm.at[pl.ds(base, per_w)], idx_v)

        def gather(j, b):
            return pltpu.make_async_copy(
                table_hbm.at[idx_v.at[pl.ds(j * ch, ch)]], rows_v.at[b], sg[b]
            )

        def put(j, b):
            return pltpu.make_async_copy(
                rows_v.at[b], out_hbm.at[pl.ds(base + j * ch, ch)], sp[b]
            )

        for b in range(nbuf):
            gather(b, b).start()

        def body(g, carry):
            j0 = g * nbuf
            for b in range(nbuf):
                gather(j0 + b, b).wait()
                put(j0 + b, b).start()
            for b in range(nbuf):
                put(j0 + b, b).wait()

                @pl.when(g + 1 < n_grp)
                def _():
                    gather(j0 + nbuf + b, b).start()

            return carry

        lax.fori_loop(0, n_grp, body, 0)

    return k(table, flat_idx)


def _sc_scatter_add(msg, dst, zeros_nd):
    e, d = msg.shape
    n_pad = zeros_nd.shape[0]
    info = plsc.get_sparse_core_info()
    nc, ns = info.num_cores, info.num_subcores
    nw = nc * ns
    per_w = e // nw
    ch = 80
    n_ch = per_w // ch
    rows_per_tile = n_pad // ns
    dst3d = dst.reshape(nw, n_ch, ch)
    mesh = plsc.VectorSubcoreMesh(core_axis_name="c", subcore_axis_name="s")

    @functools.partial(
        pl.kernel,
        mesh=mesh,
        out_type=jax.ShapeDtypeStruct((nc, n_pad, d), jnp.float32),
        scratch_types=[
            pltpu.VMEM((n_ch, ch), jnp.int32),
            pltpu.VMEM((ch, d), jnp.float32),
            pltpu.VMEM_SHARED((n_pad, d), jnp.float32),
        ],
    )
    def k(msg_hbm, dst_hbm, zeros_hbm, out_hbm, idx_v, rows_v, acc_sh):
        cid = lax.axis_index("c")
        sid = lax.axis_index("s")
        wid = sid * nc + cid
        pltpu.sync_copy(
            zeros_hbm.at[pl.ds(sid * rows_per_tile, rows_per_tile)],
            acc_sh.at[pl.ds(sid * rows_per_tile, rows_per_tile)],
        )
        plsc.subcore_barrier()

        base = wid * per_w
        pltpu.sync_copy(dst_hbm.at[wid], idx_v)

        def body(j, carry):
            pltpu.sync_copy(msg_hbm.at[pl.ds(base + j * ch, ch)], rows_v)
            pltpu.sync_copy(rows_v, acc_sh.at[idx_v.at[j]], add=True)
            return carry

        lax.fori_loop(0, n_ch, body, 0)
        plsc.subcore_barrier()
        pltpu.sync_copy(
            acc_sh.at[pl.ds(sid * rows_per_tile, rows_per_tile)],
            out_hbm.at[cid].at[pl.ds(sid * rows_per_tile, rows_per_tile)],
        )

    return k(msg, dst3d, zeros_nd)


def _ln(x, g, b):
    m = jnp.mean(x, axis=-1, keepdims=True)
    v = jnp.mean((x - m) ** 2, axis=-1, keepdims=True)
    return (x - m) * lax.rsqrt(v + 1e-5) * g + b


def _dot(a, b):
    return jnp.dot(a, b, preferred_element_type=jnp.float32)


def _edge_body(g_ref, ef_ref, Wgs_ref, Wgd_ref, Wge_ref, bg_ref,
               Ws_ref, Wep_ref, bm_ref, Wus_ref, Wud_ref, Wue_ref, bu1_ref,
               Wu2_ref, bu2_ref, ge_ref, be_ref, msg_ref, eo_ref):
    hs = g_ref[:_EBLK]
    hd = g_ref[_EBLK:]
    ef = ef_ref[...]
    gate = jax.nn.sigmoid(
        _dot(hs, Wgs_ref[...]) + _dot(hd, Wgd_ref[...])
        + _dot(ef, Wge_ref[...]) + bg_ref[...]
    )
    lin = _dot(hs, Ws_ref[...]) + _dot(ef, Wep_ref[...]) + bm_ref[...]
    msg_ref[...] = gate * lin
    u1 = jax.nn.silu(
        _dot(hs, Wus_ref[...]) + _dot(hd, Wud_ref[...])
        + _dot(ef, Wue_ref[...]) + bu1_ref[...]
    )
    eu = _dot(u1, Wu2_ref[...]) + bu2_ref[...]
    eo_ref[...] = _ln(ef + eu, ge_ref[...], be_ref[...])


_EBLK = 2000


def _tc_edge(gathered, ef, blk0, e_seg, eo_prev, Wg, bg, Ws, Wep, bm, Wu1,
             bu1, Wu2, bu2, ge, be):
    e, d = ef.shape
    grid = (e_seg // _EBLK,)
    row_spec = pl.BlockSpec((_EBLK, d), lambda i: (i, 0))
    ef_spec = pl.BlockSpec((_EBLK, d), lambda i, b=blk0: (b + i, 0))
    g_spec = pl.BlockSpec((2 * _EBLK, d), lambda i: (i, 0))
    full = lambda a: pl.BlockSpec(a.shape, lambda i: (0,) * a.ndim)
    ws = [Wg[:d], Wg[d:2 * d], Wg[2 * d:], bg, Ws, Wep, bm,
          Wu1[:d], Wu1[d:2 * d], Wu1[2 * d:], bu1, Wu2, bu2, ge, be]
    in_specs = [g_spec, ef_spec] + [full(w) for w in ws]
    args = [gathered, ef] + ws
    aliases = {}
    if eo_prev is not None:
        in_specs.append(pl.BlockSpec(memory_space=pl.ANY))
        args.append(eo_prev)
        aliases = {len(args) - 1: 1}
    body = _edge_body if eo_prev is None else (
        lambda *refs: _edge_body(*refs[:17], *refs[18:]))
    return pl.pallas_call(
        body,
        grid=grid,
        in_specs=in_specs,
        out_specs=[row_spec, ef_spec],
        out_shape=[jax.ShapeDtypeStruct((e_seg, d), jnp.float32),
                   jax.ShapeDtypeStruct((e, d), jnp.float32)],
        input_output_aliases=aliases,
    )(*args)


def _make_node_body(n_parts):
    def body(*refs):
        nf_ref = refs[0]
        parts = refs[1:1 + n_parts]
        (W1a_ref, W1b_ref, b1_ref, W2_ref, b2_ref, gn_ref, bn_ref,
         out_ref) = refs[1 + n_parts:]
        nf = nf_ref[...]
        agg = parts[0][0] + parts[0][1]
        for p in parts[1:]:
            agg = agg + p[0] + p[1]
        h1 = jax.nn.silu(
            jnp.dot(nf, W1a_ref[...], preferred_element_type=jnp.float32)
            + jnp.dot(agg, W1b_ref[...], preferred_element_type=jnp.float32)
            + b1_ref[...]
        )
        h = (jnp.dot(h1, W2_ref[...], preferred_element_type=jnp.float32)
             + b2_ref[...])
        out_ref[...] = _ln(nf + h, gn_ref[...], bn_ref[...])

    return body


def _tc_node(nf, parts, W1a, W1b, b1, W2, b2, gn, bn):
    n, d = nf.shape
    bn_blk = 1000
    grid = (n // bn_blk,)
    row_spec = pl.BlockSpec((bn_blk, d), lambda i: (i, 0))
    p_spec = pl.BlockSpec((2, bn_blk, d), lambda i: (0, i, 0))
    full = lambda a: pl.BlockSpec(a.shape, lambda i: (0,) * a.ndim)
    ws = [W1a, W1b, b1, W2, b2, gn, bn]
    return pl.pallas_call(
        _make_node_body(len(parts)),
        grid=grid,
        in_specs=[row_spec] + [p_spec] * len(parts) + [full(w) for w in ws],
        out_specs=row_spec,
        out_shape=jax.ShapeDtypeStruct((n, d), jnp.float32),
    )(nf, *parts, *ws)


def kernel(node_feats, edge_index, edge_feats, Ws, bs, Wep, bep, Wg, bg,
           Wn1, bn1, Wn2, bn2, Wu1, bu1, Wu2, bu2, g_node, b_node,
           g_edge, b_edge):
    n, d = node_feats.shape
    e = edge_feats.shape[0]

    nblk = e // _EBLK
    flat_idx = (edge_index.astype(jnp.int32)
                .reshape(2, nblk, _EBLK)
                .transpose(1, 0, 2)
                .reshape(2 * e))
    dst = edge_index[1].astype(jnp.int32)
    n_pad = ((n + 127) // 128) * 128
    zeros = jnp.zeros((n_pad, d), jnp.float32)
    row = lambda v: v.reshape(1, d)

    n_seg = 5
    e_seg = e // n_seg
    edge_out, parts = None, []
    for s in range(n_seg):
        gathered = _sc_gather(node_feats,
                              flat_idx[2 * e_seg * s:2 * e_seg * (s + 1)])
        msg_s, edge_out = _tc_edge(
            gathered, edge_feats, s * (e_seg // _EBLK), e_seg, edge_out,
            Wg, row(bg), Ws, Wep, row(bs + bep),
            Wu1, row(bu1), Wu2, row(bu2), row(g_edge), row(b_edge),
        )
        p = _sc_scatter_add(msg_s, dst[e_seg * s:e_seg * (s + 1)], zeros)
        parts.append(p)
    node_out = _tc_node(
        node_feats, parts,
        Wn1[:d], Wn1[d:], row(bn1), Wn2, row(bn2),
        row(g_node), row(b_node),
    )
    return (node_out, edge_out)
